# baseline (speedup 1.0000x reference)
import jax
import jax.numpy as jnp
from jax.experimental import pallas as pl

EMBED_DIM = 2048
NUM_EXPERTS = 16
N_TOKENS = 16384
BLK = 512


def _gate_body(x_ref, w_ref, b_ref, gates_ref, idx_ref):
    x = x_ref[...]
    w = w_ref[...]
    logits = jax.lax.dot_general(
        x, w, (((1,), (1,)), ((), ())),
        preferred_element_type=jnp.float32)
    logits = logits + b_ref[...]

    cols = jax.lax.broadcasted_iota(jnp.int32, logits.shape, 1)
    m1 = jnp.max(logits, axis=1, keepdims=True)
    i1 = jnp.min(jnp.where(logits == m1, cols, NUM_EXPERTS),
                 axis=1, keepdims=True)
    masked = jnp.where(cols == i1, -jnp.inf, logits)
    m2 = jnp.max(masked, axis=1, keepdims=True)
    i2 = jnp.min(jnp.where(masked == m2, cols, NUM_EXPERTS),
                 axis=1, keepdims=True)

    e2 = jnp.exp(m2 - m1)
    denom = 1.0 + e2
    g1 = 1.0 / denom
    g2 = e2 / denom
    gates_ref[...] = jnp.concatenate([g1, g2], axis=1)
    idx_ref[...] = jnp.concatenate([i1, i2], axis=1)


def kernel(x, W, b):
    grid = (N_TOKENS // BLK,)
    gates, idx = pl.pallas_call(
        _gate_body,
        grid=grid,
        in_specs=[
            pl.BlockSpec((BLK, EMBED_DIM), lambda i: (i, 0)),
            pl.BlockSpec((NUM_EXPERTS, EMBED_DIM), lambda i: (0, 0)),
            pl.BlockSpec((1, NUM_EXPERTS), lambda i: (0, 0)),
        ],
        out_specs=[
            pl.BlockSpec((BLK, 2), lambda i: (i, 0)),
            pl.BlockSpec((BLK, 2), lambda i: (i, 0)),
        ],
        out_shape=[
            jax.ShapeDtypeStruct((N_TOKENS, 2), jnp.float32),
            jax.ShapeDtypeStruct((N_TOKENS, 2), jnp.int32),
        ],
    )(x, W, b.reshape(1, NUM_EXPERTS))
    return gates, idx


# speedup vs baseline: 1.1959x; 1.1959x over previous
import jax
import jax.numpy as jnp
from jax.experimental import pallas as pl

EMBED_DIM = 2048
NUM_EXPERTS = 16
N_TOKENS = 16384
BLK = 2048


def _gate_body(x_ref, w_ref, b_ref, gates_ref, idx_ref):
    x = x_ref[...]
    w = w_ref[...]
    logits = jax.lax.dot_general(
        x, w, (((1,), (1,)), ((), ())),
        preferred_element_type=jnp.float32)
    logits = logits + b_ref[...]

    cols = jax.lax.broadcasted_iota(jnp.int32, logits.shape, 1)
    m1 = jnp.max(logits, axis=1, keepdims=True)
    i1 = jnp.min(jnp.where(logits == m1, cols, NUM_EXPERTS),
                 axis=1, keepdims=True)
    masked = jnp.where(cols == i1, -jnp.inf, logits)
    m2 = jnp.max(masked, axis=1, keepdims=True)
    i2 = jnp.min(jnp.where(masked == m2, cols, NUM_EXPERTS),
                 axis=1, keepdims=True)

    e2 = jnp.exp(m2 - m1)
    denom = 1.0 + e2
    g1 = 1.0 / denom
    g2 = e2 / denom
    gates_ref[...] = jnp.concatenate([g1, g2], axis=1)
    idx_ref[...] = jnp.concatenate([i1, i2], axis=1)


def kernel(x, W, b):
    grid = (N_TOKENS // BLK,)
    gates, idx = pl.pallas_call(
        _gate_body,
        grid=grid,
        in_specs=[
            pl.BlockSpec((BLK, EMBED_DIM), lambda i: (i, 0)),
            pl.BlockSpec((NUM_EXPERTS, EMBED_DIM), lambda i: (0, 0)),
            pl.BlockSpec((1, NUM_EXPERTS), lambda i: (0, 0)),
        ],
        out_specs=[
            pl.BlockSpec((BLK, 2), lambda i: (i, 0)),
            pl.BlockSpec((BLK, 2), lambda i: (i, 0)),
        ],
        out_shape=[
            jax.ShapeDtypeStruct((N_TOKENS, 2), jnp.float32),
            jax.ShapeDtypeStruct((N_TOKENS, 2), jnp.int32),
        ],
    )(x, W, b.reshape(1, NUM_EXPERTS))
    return gates, idx
